# baseline (speedup 1.0000x reference)
import functools

import jax
import jax.numpy as jnp
from jax import lax
from jax.experimental import pallas as pl
from jax.experimental.pallas import tpu as pltpu
from jax.experimental.pallas import tpu_sc as plsc

D_LFA = 128
D_FEAT = 128
K = 32

NC = 2
NS = 16
L = 16
NW = NC * NS
NPW = 320
G = 4
ROWS = G * K
GROUPS = NPW // G
N_PAD = NW * NPW


def _sc_gather_mean(features2d, idx_grouped):
    mesh = plsc.VectorSubcoreMesh(
        core_axis_name="c", subcore_axis_name="s", num_cores=NC, num_subcores=NS
    )

    @functools.partial(
        pl.kernel,
        out_type=jax.ShapeDtypeStruct((N_PAD, D_FEAT), jnp.float32),
        mesh=mesh,
        compiler_params=pltpu.CompilerParams(
            needs_layout_passes=False, use_tc_tiling_on_sc=False),
        scratch_types=[
            pltpu.VMEM((GROUPS, ROWS), jnp.int32),
            pltpu.VMEM((ROWS, D_FEAT // 2), jnp.float32),
            pltpu.VMEM((ROWS, D_FEAT // 2), jnp.float32),
            pltpu.VMEM((G, D_FEAT), jnp.float32),
            pltpu.VMEM((G, D_FEAT), jnp.float32),
            pltpu.VMEM_SHARED(features2d.shape, jnp.float32),
            pltpu.SemaphoreType.DMA,
            pltpu.SemaphoreType.DMA,
            pltpu.SemaphoreType.DMA,
            pltpu.SemaphoreType.DMA,
        ],
    )
    def gather_mean(feat_hbm, idx_hbm, out_hbm, idx_v, buf0, buf1, acc0, acc1,
                    feat_sh, sem0, sem1, semo0, semo1):
        wid = lax.axis_index("s") * NC + lax.axis_index("c")
        sid = lax.axis_index("s")

        n_tab = features2d.shape[0]
        rows_per_tile = (n_tab // NS) // 8 * 8
        rem = n_tab - rows_per_tile * NS
        stage = pl.ds(sid * rows_per_tile, rows_per_tile)
        pltpu.sync_copy(feat_hbm.at[stage], feat_sh.at[stage])
        if rem:
            @pl.when(sid == 0)
            def _():
                tail = pl.ds(NS * rows_per_tile, rem)
                pltpu.sync_copy(feat_hbm.at[tail], feat_sh.at[tail])

        pltpu.sync_copy(idx_hbm.at[pl.ds(wid * GROUPS, GROUPS)], idx_v)
        plsc.subcore_barrier()
        pltpu.async_copy(feat_sh.at[idx_v.at[0]], buf0, sem0)

        nchunks = D_FEAT // (2 * L)

        def process(g, buf, acc, semo):
            @pl.when(g >= 2)
            def _():
                pltpu.make_async_copy(
                    acc, out_hbm.at[pl.ds(wid * NPW + g * G, G)], semo).wait()

            def node(i, carry):
                base = i * K
                acc_e = [jnp.zeros((L,), jnp.float32) for _ in range(nchunks)]
                acc_o = [jnp.zeros((L,), jnp.float32) for _ in range(nchunks)]
                hi_mask = jnp.full((L,), -65536, jnp.int32)
                for kk in range(K):
                    for c in range(nchunks):
                        x32 = buf[base + kk, pl.ds(c * L, L)]
                        w32 = plsc.bitcast(x32, jnp.int32)
                        a = plsc.bitcast(w32 << 16, jnp.float32)
                        bvec = plsc.bitcast(w32 & hi_mask, jnp.float32)
                        acc_e[c] = acc_e[c] + a
                        acc_o[c] = acc_o[c] + bvec
                rowv = jnp.full((L,), i, jnp.int32)
                lanes = lax.iota(jnp.int32, L)
                for c in range(nchunks):
                    col_e = 2 * lanes + c * 2 * L
                    plsc.store_scatter(acc, [rowv, col_e],
                                       acc_e[c] * (1.0 / K))
                    plsc.store_scatter(acc, [rowv, col_e + 1],
                                       acc_o[c] * (1.0 / K))
                return carry
            lax.fori_loop(0, G, node, 0)
            pltpu.async_copy(acc, out_hbm.at[pl.ds(wid * NPW + g * G, G)], semo)

        def body(gg, carry):
            g0 = 2 * gg
            g1 = g0 + 1
            pltpu.async_copy(feat_sh.at[idx_v.at[g1]], buf1, sem1)
            pltpu.make_async_copy(feat_sh.at[idx_v.at[g0]], buf0, sem0).wait()
            process(g0, buf0, acc0, semo0)

            @pl.when(g1 + 1 < GROUPS)
            def _():
                pltpu.async_copy(feat_sh.at[idx_v.at[g1 + 1]], buf0, sem0)

            pltpu.make_async_copy(feat_sh.at[idx_v.at[g1]], buf1, sem1).wait()
            process(g1, buf1, acc1, semo1)
            return carry

        lax.fori_loop(0, GROUPS // 2, body, 0)
        pltpu.make_async_copy(
            acc0, out_hbm.at[pl.ds(wid * NPW, G)], semo0).wait()
        pltpu.make_async_copy(
            acc1, out_hbm.at[pl.ds(wid * NPW, G)], semo1).wait()

    return gather_mean(features2d, idx_grouped)


KC = 8


def _tc_geom_mlp(geom2, w_bd, b_tiled):
    n = geom2.shape[0]
    nb = 1000
    grid = (n // nb, K // KC)

    def body(g_ref, w_ref, b_ref, o_ref):
        c = pl.program_id(1)
        t = jnp.dot(g_ref[...], w_ref[...], preferred_element_type=jnp.float32)
        t = t + b_ref[...]
        t = jnp.where(t >= 0, t, 0.1 * t)
        s = t[:, 0:D_LFA]
        for j in range(1, KC):
            s = s + t[:, j * D_LFA:(j + 1) * D_LFA]
        s = s * (1.0 / K)

        @pl.when(c == 0)
        def _():
            o_ref[...] = s

        @pl.when(c > 0)
        def _():
            o_ref[...] = o_ref[...] + s

    return pl.pallas_call(
        body,
        grid=grid,
        in_specs=[
            pl.BlockSpec((nb, K * 4), lambda i, c: (i, 0)),
            pl.BlockSpec((K * 4, KC * D_LFA), lambda i, c: (0, c)),
            pl.BlockSpec((1, KC * D_LFA), lambda i, c: (0, c)),
        ],
        out_specs=pl.BlockSpec((nb, D_LFA), lambda i, c: (i, 0)),
        out_shape=jax.ShapeDtypeStruct((n, D_LFA), jnp.float32),
    )(geom2, w_bd, b_tiled)


def kernel(features, geom_features, neighbor_indices, W, b):
    bsz, n, k_ = neighbor_indices.shape
    f2b = features.reshape(n, D_FEAT).astype(jnp.bfloat16)
    f2 = jax.lax.bitcast_convert_type(
        f2b.reshape(n, D_FEAT // 2, 2), jnp.float32)
    g2 = geom_features.reshape(n, k_ * 4)
    idx = neighbor_indices.reshape(n * k_).astype(jnp.int32)
    idx_p = jnp.zeros((N_PAD * k_,), jnp.int32).at[: n * k_].set(idx)
    idx_grouped = idx_p.reshape(N_PAD // G, ROWS)

    part_b = _sc_gather_mean(f2, idx_grouped)[:n]

    w_bd = jax.scipy.linalg.block_diag(*([W] * k_))
    b_tiled = jnp.tile(b, (k_,)).reshape(1, k_ * D_LFA)
    part_a = _tc_geom_mlp(g2, w_bd, b_tiled)

    out = jnp.concatenate([part_a, part_b], axis=-1)
    return out.reshape(bsz, n, D_LFA + D_FEAT)


# speedup vs baseline: 1.3669x; 1.3669x over previous
import functools

import jax
import jax.numpy as jnp
from jax import lax
from jax.experimental import pallas as pl
from jax.experimental.pallas import tpu as pltpu
from jax.experimental.pallas import tpu_sc as plsc

D_LFA = 128
D_FEAT = 128
K = 32

NC = 2
NS = 16
L = 16
NW = NC * NS
NPW = 320
G = 4
ROWS = G * K
GROUPS = NPW // G
N_PAD = NW * NPW


def _sc_gather_mean(features2d, idx_grouped):
    mesh = plsc.VectorSubcoreMesh(
        core_axis_name="c", subcore_axis_name="s", num_cores=NC, num_subcores=NS
    )

    @functools.partial(
        pl.kernel,
        out_type=jax.ShapeDtypeStruct((N_PAD, D_FEAT), jnp.float32),
        mesh=mesh,
        compiler_params=pltpu.CompilerParams(
            needs_layout_passes=False, use_tc_tiling_on_sc=False),
        scratch_types=[
            pltpu.VMEM((GROUPS, ROWS), jnp.int32),
            pltpu.VMEM((ROWS, D_FEAT // 2), jnp.int32),
            pltpu.VMEM((ROWS, D_FEAT // 2), jnp.int32),
            pltpu.VMEM((G, D_FEAT), jnp.float32),
            pltpu.VMEM((G, D_FEAT), jnp.float32),
            pltpu.VMEM_SHARED(features2d.shape, jnp.int32),
            pltpu.SemaphoreType.DMA,
            pltpu.SemaphoreType.DMA,
            pltpu.SemaphoreType.DMA,
            pltpu.SemaphoreType.DMA,
        ],
    )
    def gather_mean(feat_hbm, idx_hbm, out_hbm, idx_v, buf0, buf1, acc0, acc1,
                    feat_sh, sem0, sem1, semo0, semo1):
        wid = lax.axis_index("s") * NC + lax.axis_index("c")
        sid = lax.axis_index("s")

        n_tab = features2d.shape[0]
        rows_per_tile = (n_tab // NS) // 8 * 8
        rem = n_tab - rows_per_tile * NS
        stage = pl.ds(sid * rows_per_tile, rows_per_tile)
        pltpu.sync_copy(feat_hbm.at[stage], feat_sh.at[stage])
        if rem:
            @pl.when(sid == 0)
            def _():
                tail = pl.ds(NS * rows_per_tile, rem)
                pltpu.sync_copy(feat_hbm.at[tail], feat_sh.at[tail])

        pltpu.sync_copy(idx_hbm.at[pl.ds(wid * GROUPS, GROUPS)], idx_v)
        plsc.subcore_barrier()
        pltpu.async_copy(feat_sh.at[idx_v.at[0]], buf0, sem0)

        nchunks = D_FEAT // (2 * L)

        def process(g, buf, acc, semo):
            @pl.when(g >= 2)
            def _():
                pltpu.make_async_copy(
                    acc, out_hbm.at[pl.ds(wid * NPW + g * G, G)], semo).wait()

            def node(i, carry):
                base = i * K
                acc_e = [jnp.zeros((L,), jnp.float32) for _ in range(nchunks)]
                acc_o = [jnp.zeros((L,), jnp.float32) for _ in range(nchunks)]
                hi_mask = jnp.full((L,), -65536, jnp.int32)
                for kk in range(K):
                    for c in range(nchunks):
                        w32 = buf[base + kk, pl.ds(c * L, L)]
                        a = plsc.bitcast(w32 << 16, jnp.float32)
                        bvec = plsc.bitcast(w32 & hi_mask, jnp.float32)
                        acc_e[c] = acc_e[c] + a
                        acc_o[c] = acc_o[c] + bvec
                for c in range(nchunks):
                    acc[i, pl.ds(c * L, L)] = acc_e[c] * (1.0 / K)
                    acc[i, pl.ds(D_FEAT // 2 + c * L, L)] = acc_o[c] * (1.0 / K)
                return carry
            lax.fori_loop(0, G, node, 0)
            pltpu.async_copy(acc, out_hbm.at[pl.ds(wid * NPW + g * G, G)], semo)

        def body(gg, carry):
            g0 = 2 * gg
            g1 = g0 + 1
            pltpu.async_copy(feat_sh.at[idx_v.at[g1]], buf1, sem1)
            pltpu.make_async_copy(feat_sh.at[idx_v.at[g0]], buf0, sem0).wait()
            process(g0, buf0, acc0, semo0)

            @pl.when(g1 + 1 < GROUPS)
            def _():
                pltpu.async_copy(feat_sh.at[idx_v.at[g1 + 1]], buf0, sem0)

            pltpu.make_async_copy(feat_sh.at[idx_v.at[g1]], buf1, sem1).wait()
            process(g1, buf1, acc1, semo1)
            return carry

        lax.fori_loop(0, GROUPS // 2, body, 0)
        pltpu.make_async_copy(
            acc0, out_hbm.at[pl.ds(wid * NPW, G)], semo0).wait()
        pltpu.make_async_copy(
            acc1, out_hbm.at[pl.ds(wid * NPW, G)], semo1).wait()

    return gather_mean(features2d, idx_grouped)


KC = 8


def _tc_geom_mlp(geom2, w_bd, b_tiled):
    n = geom2.shape[0]
    nb = 1000
    grid = (n // nb, K // KC)

    def body(g_ref, w_ref, b_ref, o_ref):
        c = pl.program_id(1)
        t = jnp.dot(g_ref[...], w_ref[...], preferred_element_type=jnp.float32)
        t = t + b_ref[...]
        t = jnp.where(t >= 0, t, 0.1 * t)
        s = t[:, 0:D_LFA]
        for j in range(1, KC):
            s = s + t[:, j * D_LFA:(j + 1) * D_LFA]
        s = s * (1.0 / K)

        @pl.when(c == 0)
        def _():
            o_ref[...] = s

        @pl.when(c > 0)
        def _():
            o_ref[...] = o_ref[...] + s

    return pl.pallas_call(
        body,
        grid=grid,
        in_specs=[
            pl.BlockSpec((nb, K * 4), lambda i, c: (i, 0)),
            pl.BlockSpec((K * 4, KC * D_LFA), lambda i, c: (0, c)),
            pl.BlockSpec((1, KC * D_LFA), lambda i, c: (0, c)),
        ],
        out_specs=pl.BlockSpec((nb, D_LFA), lambda i, c: (i, 0)),
        out_shape=jax.ShapeDtypeStruct((n, D_LFA), jnp.float32),
    )(geom2, w_bd, b_tiled)


def kernel(features, geom_features, neighbor_indices, W, b):
    bsz, n, k_ = neighbor_indices.shape
    u = jax.lax.bitcast_convert_type(
        features.reshape(n, D_FEAT), jnp.uint32)
    half = D_FEAT // 2
    lo = (u[:, :half] + jnp.uint32(0x8000)) >> 16
    hi = (u[:, half:] + jnp.uint32(0x8000)) & jnp.uint32(0xFFFF0000)
    f2 = jax.lax.bitcast_convert_type(lo | hi, jnp.int32)
    g2 = geom_features.reshape(n, k_ * 4)
    idx = neighbor_indices.reshape(n * k_).astype(jnp.int32)
    idx_p = jnp.zeros((N_PAD * k_,), jnp.int32).at[: n * k_].set(idx)
    idx_grouped = idx_p.reshape(N_PAD // G, ROWS)

    part_b = _sc_gather_mean(f2, idx_grouped)[:n]

    eye = jnp.eye(k_, dtype=jnp.float32)
    w_bd = (eye[:, None, :, None] * W[None, :, None, :]).reshape(
        k_ * 4, k_ * D_LFA)
    b_tiled = jnp.tile(b, (k_,)).reshape(1, k_ * D_LFA)
    part_a = _tc_geom_mlp(g2, w_bd, b_tiled)

    out = jnp.concatenate([part_a, part_b], axis=-1)
    return out.reshape(bsz, n, D_LFA + D_FEAT)


# speedup vs baseline: 1.4371x; 1.0514x over previous
import functools

import jax
import jax.numpy as jnp
from jax import lax
from jax.experimental import pallas as pl
from jax.experimental.pallas import tpu as pltpu
from jax.experimental.pallas import tpu_sc as plsc

D_LFA = 128
D_FEAT = 128
K = 32

NC = 2
NS = 16
L = 16
NW = NC * NS
NPW = 320
G = 4
ROWS = G * K
GROUPS = NPW // G
N_PAD = NW * NPW


def _sc_gather_mean(features2d, idx_grouped, n_out):
    mesh = plsc.VectorSubcoreMesh(
        core_axis_name="c", subcore_axis_name="s", num_cores=NC, num_subcores=NS
    )

    @functools.partial(
        pl.kernel,
        out_type=jax.ShapeDtypeStruct((n_out, D_FEAT), jnp.float32),
        mesh=mesh,
        compiler_params=pltpu.CompilerParams(
            needs_layout_passes=False, use_tc_tiling_on_sc=False),
        scratch_types=[
            pltpu.VMEM((GROUPS, ROWS), jnp.int32),
            pltpu.VMEM((ROWS, D_FEAT // 2), jnp.int32),
            pltpu.VMEM((ROWS, D_FEAT // 2), jnp.int32),
            pltpu.VMEM((G, D_FEAT), jnp.float32),
            pltpu.VMEM((G, D_FEAT), jnp.float32),
            pltpu.VMEM_SHARED(features2d.shape, jnp.int32),
            pltpu.SemaphoreType.DMA,
            pltpu.SemaphoreType.DMA,
            pltpu.SemaphoreType.DMA,
            pltpu.SemaphoreType.DMA,
        ],
    )
    def gather_mean(feat_hbm, idx_hbm, out_hbm, idx_v, buf0, buf1, acc0, acc1,
                    feat_sh, sem0, sem1, semo0, semo1):
        wid = lax.axis_index("s") * NC + lax.axis_index("c")
        sid = lax.axis_index("s")

        n_tab = features2d.shape[0]
        rows_per_tile = (n_tab // NS) // 8 * 8
        rem = n_tab - rows_per_tile * NS
        stage = pl.ds(sid * rows_per_tile, rows_per_tile)
        pltpu.sync_copy(feat_hbm.at[stage], feat_sh.at[stage])
        if rem:
            @pl.when(sid == 0)
            def _():
                tail = pl.ds(NS * rows_per_tile, rem)
                pltpu.sync_copy(feat_hbm.at[tail], feat_sh.at[tail])

        last_groups = (n_out - (NW - 1) * NPW) // G
        @pl.when(wid < NW - 1)
        def _():
            pltpu.sync_copy(idx_hbm.at[pl.ds(wid * GROUPS, GROUPS)], idx_v)

        @pl.when(wid == NW - 1)
        def _():
            pltpu.sync_copy(idx_hbm.at[pl.ds(wid * GROUPS, last_groups)],
                            idx_v.at[pl.ds(0, last_groups)])

        plsc.subcore_barrier()
        pltpu.async_copy(feat_sh.at[idx_v.at[0]], buf0, sem0)

        nchunks = D_FEAT // (2 * L)

        def process(g, buf, acc, semo):
            @pl.when(g >= 2)
            def _():
                pltpu.make_async_copy(
                    acc, out_hbm.at[pl.ds(wid * NPW + g * G, G)], semo).wait()

            def node(i, carry):
                base = i * K
                acc_e = [jnp.zeros((L,), jnp.float32) for _ in range(nchunks)]
                acc_o = [jnp.zeros((L,), jnp.float32) for _ in range(nchunks)]
                hi_mask = jnp.full((L,), -65536, jnp.int32)
                for kk in range(K):
                    for c in range(nchunks):
                        w32 = buf[base + kk, pl.ds(c * L, L)]
                        a = plsc.bitcast(w32 << 16, jnp.float32)
                        bvec = plsc.bitcast(w32 & hi_mask, jnp.float32)
                        acc_e[c] = acc_e[c] + a
                        acc_o[c] = acc_o[c] + bvec
                for c in range(nchunks):
                    acc[i, pl.ds(c * L, L)] = acc_e[c] * (1.0 / K)
                    acc[i, pl.ds(D_FEAT // 2 + c * L, L)] = acc_o[c] * (1.0 / K)
                return carry
            lax.fori_loop(0, G, node, 0)
            pltpu.async_copy(acc, out_hbm.at[pl.ds(wid * NPW + g * G, G)], semo)

        def body(gg, carry):
            g0 = 2 * gg
            g1 = g0 + 1
            pltpu.async_copy(feat_sh.at[idx_v.at[g1]], buf1, sem1)
            pltpu.make_async_copy(feat_sh.at[idx_v.at[g0]], buf0, sem0).wait()
            process(g0, buf0, acc0, semo0)

            @pl.when(g1 + 1 < my_groups)
            def _():
                pltpu.async_copy(feat_sh.at[idx_v.at[g1 + 1]], buf0, sem0)

            pltpu.make_async_copy(feat_sh.at[idx_v.at[g1]], buf1, sem1).wait()
            process(g1, buf1, acc1, semo1)
            return carry

        my_groups = jnp.where(wid == NW - 1, last_groups, GROUPS)
        lax.fori_loop(0, my_groups // 2, body, 0)
        pltpu.make_async_copy(
            acc0, out_hbm.at[pl.ds(wid * NPW, G)], semo0).wait()
        pltpu.make_async_copy(
            acc1, out_hbm.at[pl.ds(wid * NPW, G)], semo1).wait()

    return gather_mean(features2d, idx_grouped)


KC = 8


def _tc_geom_mlp(geom2, w_bd, b_tiled):
    n = geom2.shape[0]
    nb = 1000
    grid = (n // nb, K // KC)

    def body(g_ref, w_ref, b_ref, o_ref):
        c = pl.program_id(1)
        t = jnp.dot(g_ref[...].astype(jnp.bfloat16), w_ref[...],
                    preferred_element_type=jnp.float32)
        t = t + b_ref[...]
        t = jnp.where(t >= 0, t, 0.1 * t)
        s = t[:, 0:D_LFA]
        for j in range(1, KC):
            s = s + t[:, j * D_LFA:(j + 1) * D_LFA]
        s = s * (1.0 / K)

        @pl.when(c == 0)
        def _():
            o_ref[...] = s

        @pl.when(c > 0)
        def _():
            o_ref[...] = o_ref[...] + s

    return pl.pallas_call(
        body,
        grid=grid,
        in_specs=[
            pl.BlockSpec((nb, K * 4), lambda i, c: (i, 0)),
            pl.BlockSpec((K * 4, KC * D_LFA), lambda i, c: (0, c)),
            pl.BlockSpec((1, KC * D_LFA), lambda i, c: (0, c)),
        ],
        out_specs=pl.BlockSpec((nb, D_LFA), lambda i, c: (i, 0)),
        out_shape=jax.ShapeDtypeStruct((n, D_LFA), jnp.float32),
    )(geom2, w_bd, b_tiled)


def kernel(features, geom_features, neighbor_indices, W, b):
    bsz, n, k_ = neighbor_indices.shape
    u = jax.lax.bitcast_convert_type(
        features.reshape(n, D_FEAT), jnp.uint32)
    half = D_FEAT // 2
    lo = (u[:, :half] + jnp.uint32(0x8000)) >> 16
    hi = (u[:, half:] + jnp.uint32(0x8000)) & jnp.uint32(0xFFFF0000)
    f2 = jax.lax.bitcast_convert_type(lo | hi, jnp.int32)
    g2 = geom_features.reshape(n, k_ * 4)
    idx_grouped = neighbor_indices.reshape(
        n * k_ // ROWS, ROWS).astype(jnp.int32)

    part_b = _sc_gather_mean(f2, idx_grouped, n)

    eye = jnp.eye(k_, dtype=jnp.float32)
    w_bd = (eye[:, None, :, None] * W[None, :, None, :]).reshape(
        k_ * 4, k_ * D_LFA).astype(jnp.bfloat16)
    b_tiled = jnp.tile(b, (k_,)).reshape(1, k_ * D_LFA)
    part_a = _tc_geom_mlp(g2, w_bd, b_tiled)

    out = jnp.concatenate([part_a, part_b], axis=-1)
    return out.reshape(bsz, n, D_LFA + D_FEAT)


# speedup vs baseline: 1.4534x; 1.0113x over previous
import functools

import jax
import jax.numpy as jnp
from jax import lax
from jax.experimental import pallas as pl
from jax.experimental.pallas import tpu as pltpu
from jax.experimental.pallas import tpu_sc as plsc

D_LFA = 128
D_FEAT = 128
K = 32

NC = 2
NS = 16
L = 16
NW = NC * NS
NPW = 320
G = 4
ROWS = G * K
GROUPS = NPW // G
N_PAD = NW * NPW


def _sc_gather_mean(features2d, idx_grouped, n_out):
    mesh = plsc.VectorSubcoreMesh(
        core_axis_name="c", subcore_axis_name="s", num_cores=NC, num_subcores=NS
    )

    @functools.partial(
        pl.kernel,
        out_type=jax.ShapeDtypeStruct((n_out, D_FEAT), jnp.float32),
        mesh=mesh,
        compiler_params=pltpu.CompilerParams(
            needs_layout_passes=False, use_tc_tiling_on_sc=False),
        scratch_types=[
            pltpu.VMEM((GROUPS, ROWS), jnp.int32),
            pltpu.VMEM((ROWS, D_FEAT // 2), jnp.int32),
            pltpu.VMEM((ROWS, D_FEAT // 2), jnp.int32),
            pltpu.VMEM((G, D_FEAT), jnp.float32),
            pltpu.VMEM((G, D_FEAT), jnp.float32),
            pltpu.VMEM_SHARED(features2d.shape, jnp.int32),
            pltpu.SemaphoreType.DMA,
            pltpu.SemaphoreType.DMA,
            pltpu.SemaphoreType.DMA,
            pltpu.SemaphoreType.DMA,
        ],
    )
    def gather_mean(feat_hbm, idx_hbm, out_hbm, idx_v, buf0, buf1, acc0, acc1,
                    feat_sh, sem0, sem1, semo0, semo1):
        wid = lax.axis_index("s") * NC + lax.axis_index("c")
        sid = lax.axis_index("s")

        n_tab = features2d.shape[0]
        rows_per_tile = (n_tab // NS) // 8 * 8
        rem = n_tab - rows_per_tile * NS
        stage = pl.ds(sid * rows_per_tile, rows_per_tile)
        pltpu.sync_copy(feat_hbm.at[stage], feat_sh.at[stage])
        if rem:
            @pl.when(sid == 0)
            def _():
                tail = pl.ds(NS * rows_per_tile, rem)
                pltpu.sync_copy(feat_hbm.at[tail], feat_sh.at[tail])

        last_groups = (n_out - (NW - 1) * NPW) // G
        @pl.when(wid < NW - 1)
        def _():
            pltpu.sync_copy(idx_hbm.at[pl.ds(wid * GROUPS, GROUPS)], idx_v)

        @pl.when(wid == NW - 1)
        def _():
            pltpu.sync_copy(idx_hbm.at[pl.ds(wid * GROUPS, last_groups)],
                            idx_v.at[pl.ds(0, last_groups)])

        plsc.subcore_barrier()
        pltpu.async_copy(feat_sh.at[idx_v.at[0]], buf0, sem0)

        nchunks = D_FEAT // (2 * L)

        def process(g, buf, acc, semo):
            @pl.when(g >= 2)
            def _():
                pltpu.make_async_copy(
                    acc, out_hbm.at[pl.ds(wid * NPW + g * G, G)], semo).wait()

            def node(i, carry):
                base = i * K
                acc_e = [jnp.zeros((L,), jnp.float32) for _ in range(nchunks)]
                acc_o = [jnp.zeros((L,), jnp.float32) for _ in range(nchunks)]
                hi_mask = jnp.full((L,), -65536, jnp.int32)
                for kk in range(K):
                    for c in range(nchunks):
                        w32 = buf[base + kk, pl.ds(c * L, L)]
                        a = plsc.bitcast(w32 << 16, jnp.float32)
                        bvec = plsc.bitcast(w32 & hi_mask, jnp.float32)
                        acc_e[c] = acc_e[c] + a
                        acc_o[c] = acc_o[c] + bvec
                for c in range(nchunks):
                    acc[i, pl.ds(c * L, L)] = acc_e[c] * (1.0 / K)
                    acc[i, pl.ds(D_FEAT // 2 + c * L, L)] = acc_o[c] * (1.0 / K)
                return carry
            lax.fori_loop(0, G, node, 0)
            pltpu.async_copy(acc, out_hbm.at[pl.ds(wid * NPW + g * G, G)], semo)

        def body(gg, carry):
            g0 = 2 * gg
            g1 = g0 + 1
            pltpu.async_copy(feat_sh.at[idx_v.at[g1]], buf1, sem1)
            pltpu.make_async_copy(feat_sh.at[idx_v.at[g0]], buf0, sem0).wait()
            process(g0, buf0, acc0, semo0)

            @pl.when(g1 + 1 < my_groups)
            def _():
                pltpu.async_copy(feat_sh.at[idx_v.at[g1 + 1]], buf0, sem0)

            pltpu.make_async_copy(feat_sh.at[idx_v.at[g1]], buf1, sem1).wait()
            process(g1, buf1, acc1, semo1)
            return carry

        my_groups = jnp.where(wid == NW - 1, last_groups, GROUPS)
        lax.fori_loop(0, my_groups // 2, body, 0)
        pltpu.make_async_copy(
            acc0, out_hbm.at[pl.ds(wid * NPW, G)], semo0).wait()
        pltpu.make_async_copy(
            acc1, out_hbm.at[pl.ds(wid * NPW, G)], semo1).wait()

    return gather_mean(features2d, idx_grouped)


KC = 8


def _tc_geom_mlp(geom2, w_bd, b_tiled):
    n = geom2.shape[0]
    nb = 1000
    grid = (n // nb, K // KC)

    def body(g_ref, w_ref, b_ref, o_ref):
        c = pl.program_id(1)
        t = jnp.dot(g_ref[...], w_ref[...],
                    preferred_element_type=jnp.float32)
        t = t + b_ref[...]
        t = jnp.where(t >= 0, t, 0.1 * t)
        s = t[:, 0:D_LFA]
        for j in range(1, KC):
            s = s + t[:, j * D_LFA:(j + 1) * D_LFA]
        s = s * (1.0 / K)

        @pl.when(c == 0)
        def _():
            o_ref[...] = s

        @pl.when(c > 0)
        def _():
            o_ref[...] = o_ref[...] + s

    return pl.pallas_call(
        body,
        grid=grid,
        in_specs=[
            pl.BlockSpec((nb, K * 4), lambda i, c: (i, 0)),
            pl.BlockSpec((K * 4, KC * D_LFA), lambda i, c: (0, c)),
            pl.BlockSpec((1, KC * D_LFA), lambda i, c: (0, c)),
        ],
        out_specs=pl.BlockSpec((nb, D_LFA), lambda i, c: (i, 0)),
        out_shape=jax.ShapeDtypeStruct((n, D_LFA), jnp.float32),
    )(geom2, w_bd, b_tiled)


def kernel(features, geom_features, neighbor_indices, W, b):
    bsz, n, k_ = neighbor_indices.shape
    u = jax.lax.bitcast_convert_type(
        features.reshape(n, D_FEAT), jnp.uint32)
    half = D_FEAT // 2
    lo = (u[:, :half] + jnp.uint32(0x8000)) >> 16
    hi = (u[:, half:] + jnp.uint32(0x8000)) & jnp.uint32(0xFFFF0000)
    f2 = jax.lax.bitcast_convert_type(lo | hi, jnp.int32)
    g2 = geom_features.astype(jnp.bfloat16).reshape(n, k_ * 4)
    idx_grouped = neighbor_indices.reshape(
        n * k_ // ROWS, ROWS).astype(jnp.int32)

    part_b = _sc_gather_mean(f2, idx_grouped, n)

    eye = jnp.eye(k_, dtype=jnp.float32)
    w_bd = (eye[:, None, :, None] * W[None, :, None, :]).reshape(
        k_ * 4, k_ * D_LFA).astype(jnp.bfloat16)
    b_tiled = jnp.tile(b, (k_,)).reshape(1, k_ * D_LFA)
    part_a = _tc_geom_mlp(g2, w_bd, b_tiled)

    out = jnp.concatenate([part_a, part_b], axis=-1)
    return out.reshape(bsz, n, D_LFA + D_FEAT)


# speedup vs baseline: 1.5962x; 1.0982x over previous
import functools

import jax
import jax.numpy as jnp
from jax import lax
from jax.experimental import pallas as pl
from jax.experimental.pallas import tpu as pltpu
from jax.experimental.pallas import tpu_sc as plsc

D_LFA = 128
D_FEAT = 128
K = 32

NC = 2
NS = 16
L = 16
NW = NC * NS
NPW = 320
G = 4
ROWS = G * K
GROUPS = NPW // G
N_PAD = NW * NPW


def _sc_gather_mean(features2d, idx_grouped, n_out):
    mesh = plsc.VectorSubcoreMesh(
        core_axis_name="c", subcore_axis_name="s", num_cores=NC, num_subcores=NS
    )

    @functools.partial(
        pl.kernel,
        out_type=jax.ShapeDtypeStruct((n_out, D_FEAT), jnp.float32),
        mesh=mesh,
        compiler_params=pltpu.CompilerParams(
            needs_layout_passes=False, use_tc_tiling_on_sc=False),
        scratch_types=[
            pltpu.VMEM((GROUPS, ROWS), jnp.int32),
            pltpu.VMEM((ROWS, D_FEAT // 2), jnp.int32),
            pltpu.VMEM((ROWS, D_FEAT // 2), jnp.int32),
            pltpu.VMEM((G, D_FEAT), jnp.float32),
            pltpu.VMEM((G, D_FEAT), jnp.float32),
            pltpu.VMEM_SHARED(features2d.shape, jnp.int32),
            pltpu.SemaphoreType.DMA,
            pltpu.SemaphoreType.DMA,
            pltpu.SemaphoreType.DMA,
            pltpu.SemaphoreType.DMA,
        ],
    )
    def gather_mean(feat_hbm, idx_hbm, out_hbm, idx_v, buf0, buf1, acc0, acc1,
                    feat_sh, sem0, sem1, semo0, semo1):
        wid = lax.axis_index("s") * NC + lax.axis_index("c")
        sid = lax.axis_index("s")

        n_tab = features2d.shape[0]
        rows_per_tile = (n_tab // NS) // 8 * 8
        rem = n_tab - rows_per_tile * NS
        stage = pl.ds(sid * rows_per_tile, rows_per_tile)
        pltpu.sync_copy(feat_hbm.at[stage], feat_sh.at[stage])
        if rem:
            @pl.when(sid == 0)
            def _():
                tail = pl.ds(NS * rows_per_tile, rem)
                pltpu.sync_copy(feat_hbm.at[tail], feat_sh.at[tail])

        last_groups = (n_out - (NW - 1) * NPW) // G
        @pl.when(wid < NW - 1)
        def _():
            pltpu.sync_copy(idx_hbm.at[pl.ds(wid * GROUPS, GROUPS)], idx_v)

        @pl.when(wid == NW - 1)
        def _():
            pltpu.sync_copy(idx_hbm.at[pl.ds(wid * GROUPS, last_groups)],
                            idx_v.at[pl.ds(0, last_groups)])

        plsc.subcore_barrier()
        pltpu.async_copy(feat_sh.at[idx_v.at[0]], buf0, sem0)

        nchunks = D_FEAT // (2 * L)

        def process(g, buf, acc, semo):
            @pl.when(g >= 2)
            def _():
                pltpu.make_async_copy(
                    acc, out_hbm.at[pl.ds(wid * NPW + g * G, G)], semo).wait()

            def node(i, carry):
                base = i * K
                acc_e = [jnp.zeros((L,), jnp.float32) for _ in range(nchunks)]
                acc_o = [jnp.zeros((L,), jnp.float32) for _ in range(nchunks)]
                hi_mask = jnp.full((L,), -65536, jnp.int32)
                for kk in range(K):
                    for c in range(nchunks):
                        w32 = buf[base + kk, pl.ds(c * L, L)]
                        a = plsc.bitcast(w32 << 16, jnp.float32)
                        bvec = plsc.bitcast(w32 & hi_mask, jnp.float32)
                        acc_e[c] = acc_e[c] + a
                        acc_o[c] = acc_o[c] + bvec
                for c in range(nchunks):
                    acc[i, pl.ds(c * L, L)] = acc_e[c] * (1.0 / K)
                    acc[i, pl.ds(D_FEAT // 2 + c * L, L)] = acc_o[c] * (1.0 / K)
                return carry
            lax.fori_loop(0, G, node, 0)
            pltpu.async_copy(acc, out_hbm.at[pl.ds(wid * NPW + g * G, G)], semo)

        def body(gg, carry):
            g0 = 2 * gg
            g1 = g0 + 1
            pltpu.async_copy(feat_sh.at[idx_v.at[g1]], buf1, sem1)
            pltpu.make_async_copy(feat_sh.at[idx_v.at[g0]], buf0, sem0).wait()
            process(g0, buf0, acc0, semo0)

            @pl.when(g1 + 1 < my_groups)
            def _():
                pltpu.async_copy(feat_sh.at[idx_v.at[g1 + 1]], buf0, sem0)

            pltpu.make_async_copy(feat_sh.at[idx_v.at[g1]], buf1, sem1).wait()
            process(g1, buf1, acc1, semo1)
            return carry

        my_groups = jnp.where(wid == NW - 1, last_groups, GROUPS)
        lax.fori_loop(0, my_groups // 2, body, 0)
        pltpu.make_async_copy(
            acc0, out_hbm.at[pl.ds(wid * NPW, G)], semo0).wait()
        pltpu.make_async_copy(
            acc1, out_hbm.at[pl.ds(wid * NPW, G)], semo1).wait()

    return gather_mean(features2d, idx_grouped)


KC = 32


def _tc_geom_mlp(geom2, w_bd, b_tiled):
    n = geom2.shape[0]
    nb = 1000
    grid = (n // nb, K // KC)

    def body(g_ref, w_ref, b_ref, o_ref):
        c = pl.program_id(1)
        t = jnp.dot(g_ref[...], w_ref[...],
                    preferred_element_type=jnp.float32)
        t = t + b_ref[...]
        t = jnp.where(t >= 0, t, 0.1 * t)
        s = t[:, 0:D_LFA]
        for j in range(1, KC):
            s = s + t[:, j * D_LFA:(j + 1) * D_LFA]
        s = s * (1.0 / K)

        @pl.when(c == 0)
        def _():
            o_ref[...] = s

        @pl.when(c > 0)
        def _():
            o_ref[...] = o_ref[...] + s

    return pl.pallas_call(
        body,
        grid=grid,
        in_specs=[
            pl.BlockSpec((nb, K * 4), lambda i, c: (i, 0)),
            pl.BlockSpec((K * 4, KC * D_LFA), lambda i, c: (0, c)),
            pl.BlockSpec((1, KC * D_LFA), lambda i, c: (0, c)),
        ],
        out_specs=pl.BlockSpec((nb, D_LFA), lambda i, c: (i, 0)),
        out_shape=jax.ShapeDtypeStruct((n, D_LFA), jnp.float32),
    )(geom2, w_bd, b_tiled)


def kernel(features, geom_features, neighbor_indices, W, b):
    bsz, n, k_ = neighbor_indices.shape
    u = jax.lax.bitcast_convert_type(
        features.reshape(n, D_FEAT), jnp.uint32)
    half = D_FEAT // 2
    lo = (u[:, :half] + jnp.uint32(0x8000)) >> 16
    hi = (u[:, half:] + jnp.uint32(0x8000)) & jnp.uint32(0xFFFF0000)
    f2 = jax.lax.bitcast_convert_type(lo | hi, jnp.int32)
    g2 = geom_features.astype(jnp.bfloat16).reshape(n, k_ * 4)
    idx_grouped = neighbor_indices.reshape(
        n * k_ // ROWS, ROWS).astype(jnp.int32)

    part_b = _sc_gather_mean(f2, idx_grouped, n)

    eye = jnp.eye(k_, dtype=jnp.float32)
    w_bd = (eye[:, None, :, None] * W[None, :, None, :]).reshape(
        k_ * 4, k_ * D_LFA).astype(jnp.bfloat16)
    b_tiled = jnp.tile(b, (k_,)).reshape(1, k_ * D_LFA)
    part_a = _tc_geom_mlp(g2, w_bd, b_tiled)

    out = jnp.concatenate([part_a, part_b], axis=-1)
    return out.reshape(bsz, n, D_LFA + D_FEAT)


# speedup vs baseline: 1.6223x; 1.0164x over previous
import functools

import jax
import jax.numpy as jnp
from jax import lax
from jax.experimental import pallas as pl
from jax.experimental.pallas import tpu as pltpu
from jax.experimental.pallas import tpu_sc as plsc

D_LFA = 128
D_FEAT = 128
K = 32

NC = 2
NS = 16
L = 16
NW = NC * NS
NPW = 320
G = 4
ROWS = G * K
GROUPS = NPW // G
N_PAD = NW * NPW


def _sc_gather_mean(features2d, idx_grouped, n_out):
    mesh = plsc.VectorSubcoreMesh(
        core_axis_name="c", subcore_axis_name="s", num_cores=NC, num_subcores=NS
    )

    @functools.partial(
        pl.kernel,
        out_type=jax.ShapeDtypeStruct((n_out, D_FEAT), jnp.float32),
        mesh=mesh,
        compiler_params=pltpu.CompilerParams(
            needs_layout_passes=False, use_tc_tiling_on_sc=False),
        scratch_types=[
            pltpu.VMEM((GROUPS, ROWS), jnp.int32),
            pltpu.VMEM((ROWS, D_FEAT // 2), jnp.int32),
            pltpu.VMEM((ROWS, D_FEAT // 2), jnp.int32),
            pltpu.VMEM((G, D_FEAT), jnp.float32),
            pltpu.VMEM((G, D_FEAT), jnp.float32),
            pltpu.VMEM_SHARED(features2d.shape, jnp.int32),
            pltpu.SemaphoreType.DMA,
            pltpu.SemaphoreType.DMA,
            pltpu.SemaphoreType.DMA,
            pltpu.SemaphoreType.DMA,
        ],
    )
    def gather_mean(feat_hbm, idx_hbm, out_hbm, idx_v, buf0, buf1, acc0, acc1,
                    feat_sh, sem0, sem1, semo0, semo1):
        wid = lax.axis_index("s") * NC + lax.axis_index("c")
        sid = lax.axis_index("s")

        n_tab = features2d.shape[0]
        rows_per_tile = (n_tab // NS) // 8 * 8
        rem = n_tab - rows_per_tile * NS
        stage = pl.ds(sid * rows_per_tile, rows_per_tile)
        pltpu.sync_copy(feat_hbm.at[stage], feat_sh.at[stage])
        if rem:
            @pl.when(sid == 0)
            def _():
                tail = pl.ds(NS * rows_per_tile, rem)
                pltpu.sync_copy(feat_hbm.at[tail], feat_sh.at[tail])

        last_groups = (n_out - (NW - 1) * NPW) // G
        @pl.when(wid < NW - 1)
        def _():
            pltpu.sync_copy(idx_hbm.at[pl.ds(wid * GROUPS, GROUPS)], idx_v)

        @pl.when(wid == NW - 1)
        def _():
            pltpu.sync_copy(idx_hbm.at[pl.ds(wid * GROUPS, last_groups)],
                            idx_v.at[pl.ds(0, last_groups)])

        plsc.subcore_barrier()
        pltpu.async_copy(feat_sh.at[idx_v.at[0]], buf0, sem0)

        nchunks = D_FEAT // (2 * L)

        def process(g, buf, acc, semo):
            @pl.when(g >= 2)
            def _():
                pltpu.make_async_copy(
                    acc, out_hbm.at[pl.ds(wid * NPW + g * G, G)], semo).wait()

            def node(i, carry):
                base = i * K
                acc_e = [jnp.zeros((L,), jnp.float32) for _ in range(nchunks)]
                acc_o = [jnp.zeros((L,), jnp.float32) for _ in range(nchunks)]
                hi_mask = jnp.full((L,), -65536, jnp.int32)
                for kk in range(K):
                    for c in range(nchunks):
                        w32 = buf[base + kk, pl.ds(c * L, L)]
                        a = plsc.bitcast(w32 << 16, jnp.float32)
                        bvec = plsc.bitcast(w32 & hi_mask, jnp.float32)
                        acc_e[c] = acc_e[c] + a
                        acc_o[c] = acc_o[c] + bvec
                for c in range(nchunks):
                    acc[i, pl.ds(c * L, L)] = acc_e[c] * (1.0 / K)
                    acc[i, pl.ds(D_FEAT // 2 + c * L, L)] = acc_o[c] * (1.0 / K)
                return carry
            lax.fori_loop(0, G, node, 0)
            pltpu.async_copy(acc, out_hbm.at[pl.ds(wid * NPW + g * G, G)], semo)

        def body(gg, carry):
            g0 = 2 * gg
            g1 = g0 + 1
            pltpu.async_copy(feat_sh.at[idx_v.at[g1]], buf1, sem1)
            pltpu.make_async_copy(feat_sh.at[idx_v.at[g0]], buf0, sem0).wait()
            process(g0, buf0, acc0, semo0)

            @pl.when(g1 + 1 < my_groups)
            def _():
                pltpu.async_copy(feat_sh.at[idx_v.at[g1 + 1]], buf0, sem0)

            pltpu.make_async_copy(feat_sh.at[idx_v.at[g1]], buf1, sem1).wait()
            process(g1, buf1, acc1, semo1)
            return carry

        my_groups = jnp.where(wid == NW - 1, last_groups, GROUPS)
        lax.fori_loop(0, my_groups // 2, body, 0)
        pltpu.make_async_copy(
            acc0, out_hbm.at[pl.ds(wid * NPW, G)], semo0).wait()
        pltpu.make_async_copy(
            acc1, out_hbm.at[pl.ds(wid * NPW, G)], semo1).wait()

    return gather_mean(features2d, idx_grouped)


KC = 32


def _tc_geom_mlp(geom2, w_bd, b_tiled):
    n = geom2.shape[0]
    nb = 2000
    grid = (n // nb, K // KC)

    def body(g_ref, w_ref, b_ref, o_ref):
        c = pl.program_id(1)
        t = jnp.dot(g_ref[...], w_ref[...],
                    preferred_element_type=jnp.float32)
        t = t + b_ref[...]
        t = jnp.where(t >= 0, t, 0.1 * t)
        s = t[:, 0:D_LFA]
        for j in range(1, KC):
            s = s + t[:, j * D_LFA:(j + 1) * D_LFA]
        s = s * (1.0 / K)

        @pl.when(c == 0)
        def _():
            o_ref[...] = s

        @pl.when(c > 0)
        def _():
            o_ref[...] = o_ref[...] + s

    return pl.pallas_call(
        body,
        grid=grid,
        in_specs=[
            pl.BlockSpec((nb, K * 4), lambda i, c: (i, 0)),
            pl.BlockSpec((K * 4, KC * D_LFA), lambda i, c: (0, c)),
            pl.BlockSpec((1, KC * D_LFA), lambda i, c: (0, c)),
        ],
        out_specs=pl.BlockSpec((nb, D_LFA), lambda i, c: (i, 0)),
        out_shape=jax.ShapeDtypeStruct((n, D_LFA), jnp.float32),
    )(geom2, w_bd, b_tiled)


def kernel(features, geom_features, neighbor_indices, W, b):
    bsz, n, k_ = neighbor_indices.shape
    u = jax.lax.bitcast_convert_type(
        features.reshape(n, D_FEAT), jnp.uint32)
    half = D_FEAT // 2
    lo = (u[:, :half] + jnp.uint32(0x8000)) >> 16
    hi = (u[:, half:] + jnp.uint32(0x8000)) & jnp.uint32(0xFFFF0000)
    f2 = jax.lax.bitcast_convert_type(lo | hi, jnp.int32)
    g2 = geom_features.astype(jnp.bfloat16).reshape(n, k_ * 4)
    idx_grouped = neighbor_indices.reshape(
        n * k_ // ROWS, ROWS).astype(jnp.int32)

    part_b = _sc_gather_mean(f2, idx_grouped, n)

    eye = jnp.eye(k_, dtype=jnp.float32)
    w_bd = (eye[:, None, :, None] * W[None, :, None, :]).reshape(
        k_ * 4, k_ * D_LFA).astype(jnp.bfloat16)
    b_tiled = jnp.tile(b, (k_,)).reshape(1, k_ * D_LFA)
    part_a = _tc_geom_mlp(g2, w_bd, b_tiled)

    out = jnp.concatenate([part_a, part_b], axis=-1)
    return out.reshape(bsz, n, D_LFA + D_FEAT)
